# baseline (speedup 1.0000x reference)
import functools

import jax
import jax.numpy as jnp
from jax import lax
from jax.experimental import pallas as pl
from jax.experimental.pallas import tpu as pltpu
from jax.experimental.pallas import tpu_sc as plsc

N_COMP = 64
ROWS = 16384
NC, NS, LANES = 2, 16, 16
NW = NC * NS
ROWS_W = ROWS // NW
GRID = 16
BROWS = 16


BLK = ROWS // GRID


def _prep_body(zc_ref, conf_ref, arg_ref):
    x = zc_ref[...]
    m = jnp.max(x, axis=1, keepdims=True)
    iot = lax.broadcasted_iota(jnp.int32, (BLK, N_COMP), 1)
    a = jnp.min(jnp.where(x == m, iot, N_COMP), axis=1)
    conf_ref[...] = m.reshape(BLK // 128, 128)
    arg_ref[...] = a.reshape(BLK // 128, 128)


_prep = pl.pallas_call(
    _prep_body,
    grid=(GRID,),
    in_specs=[pl.BlockSpec((BLK, N_COMP), lambda i: (i, 0))],
    out_specs=(
        pl.BlockSpec((BLK // 128, 128), lambda i: (i, 0)),
        pl.BlockSpec((BLK // 128, 128), lambda i: (i, 0)),
    ),
    out_shape=(
        jax.ShapeDtypeStruct((128, 128), jnp.float32),
        jax.ShapeDtypeStruct((128, 128), jnp.int32),
    ),
)


@functools.partial(
    pl.kernel,
    out_type=(
        jax.ShapeDtypeStruct((NW, N_COMP), jnp.float32),
        jax.ShapeDtypeStruct((NW, LANES), jnp.float32),
    ),
    mesh=plsc.VectorSubcoreMesh(
        core_axis_name="c", subcore_axis_name="s",
        num_cores=NC, num_subcores=NS,
    ),
    scratch_types=(
        pltpu.VMEM((ROWS_W // 128, 128), jnp.float32),
        pltpu.VMEM((ROWS_W // 128, 128), jnp.int32),
        pltpu.VMEM((N_COMP,), jnp.float32),
        pltpu.VMEM((LANES,), jnp.float32),
        pltpu.SemaphoreType.DMA,
        pltpu.SemaphoreType.DMA,
    ),
    compiler_params=pltpu.CompilerParams(needs_layout_passes=False),
)
def _sc_hist(conf_hbm, arg_hbm, pop_hbm, confp_hbm,
             cbuf, abuf, pop, confv, sem_c, sem_a):
    wid = lax.axis_index("s") * NC + lax.axis_index("c")
    r4 = wid * (ROWS_W // 128)
    cpa = pltpu.async_copy(
        arg_hbm.at[pl.ds(r4, ROWS_W // 128), :], abuf, sem_a)
    cpc = pltpu.async_copy(
        conf_hbm.at[pl.ds(r4, ROWS_W // 128), :], cbuf, sem_c)

    zeros16 = jnp.zeros((LANES,), jnp.float32)
    for k in range(N_COMP // LANES):
        pop[pl.ds(k * LANES, LANES)] = zeros16
    ones16 = jnp.ones((LANES,), jnp.float32)

    cpa.wait()
    for t in range(ROWS_W // LANES):
        idx = abuf[t // 8, pl.ds((t % 8) * LANES, LANES)]
        plsc.store_scatter(pop, [idx], ones16)
    cpc.wait()
    acc = zeros16
    for t in range(ROWS_W // LANES):
        acc = acc + cbuf[t // 8, pl.ds((t % 8) * LANES, LANES)]
    confv[...] = acc
    pltpu.sync_copy(pop, pop_hbm.at[wid])
    pltpu.sync_copy(confv, confp_hbm.at[wid])


def _merge_body(pop_ref, conf_ref, np_ref, cm_ref):
    present = jnp.max(pop_ref[...], axis=0, keepdims=True)
    num_pop = jnp.sum(jnp.where(present > 0.0, 1.0, 0.0))
    np_ref[...] = num_pop.reshape(1, 1)
    cm_ref[...] = (jnp.sum(conf_ref[...]) * (1.0 / ROWS)).reshape(1, 1)


_merge = pl.pallas_call(
    _merge_body,
    out_shape=(
        jax.ShapeDtypeStruct((1, 1), jnp.float32),
        jax.ShapeDtypeStruct((1, 1), jnp.float32),
    ),
)


def kernel(z, z_cat):
    conf2, arg2 = _prep(z_cat)
    pop_part, conf_part = _sc_hist(conf2, arg2)
    num_pop, conf_mean = _merge(pop_part, conf_part)
    return (z, num_pop[0, 0], conf_mean[0, 0])


# speedup vs baseline: 1.1258x; 1.1258x over previous
import functools

import jax
import jax.numpy as jnp
from jax import lax
from jax.experimental import pallas as pl
from jax.experimental.pallas import tpu as pltpu
from jax.experimental.pallas import tpu_sc as plsc

N_COMP = 64
ROWS = 16384
NC, NS, LANES = 2, 16, 16
NW = NC * NS
ROWS_W = ROWS // NW
GRID = 4
BROWS = 16


BLK = ROWS // GRID


def _prep_body(zc_ref, conf_ref, arg_ref):
    x = zc_ref[...]
    m = jnp.max(x, axis=1, keepdims=True)
    iot = lax.broadcasted_iota(jnp.int32, (BLK, N_COMP), 1)
    a = jnp.min(jnp.where(x == m, iot, N_COMP), axis=1)
    conf_ref[...] = m.reshape(BLK // 128, 128)
    arg_ref[...] = a.reshape(BLK // 128, 128)


_prep = pl.pallas_call(
    _prep_body,
    grid=(GRID,),
    in_specs=[pl.BlockSpec((BLK, N_COMP), lambda i: (i, 0))],
    out_specs=(
        pl.BlockSpec((BLK // 128, 128), lambda i: (i, 0)),
        pl.BlockSpec((BLK // 128, 128), lambda i: (i, 0)),
    ),
    out_shape=(
        jax.ShapeDtypeStruct((128, 128), jnp.float32),
        jax.ShapeDtypeStruct((128, 128), jnp.int32),
    ),
)


@functools.partial(
    pl.kernel,
    out_type=(
        jax.ShapeDtypeStruct((NW, N_COMP), jnp.float32),
        jax.ShapeDtypeStruct((NW, LANES), jnp.float32),
    ),
    mesh=plsc.VectorSubcoreMesh(
        core_axis_name="c", subcore_axis_name="s",
        num_cores=NC, num_subcores=NS,
    ),
    scratch_types=(
        pltpu.VMEM((ROWS_W // 128, 128), jnp.float32),
        pltpu.VMEM((ROWS_W // 128, 128), jnp.int32),
        pltpu.VMEM((N_COMP,), jnp.float32),
        pltpu.VMEM((LANES,), jnp.float32),
        pltpu.SemaphoreType.DMA,
        pltpu.SemaphoreType.DMA,
    ),
    compiler_params=pltpu.CompilerParams(needs_layout_passes=False),
)
def _sc_hist(conf_hbm, arg_hbm, pop_hbm, confp_hbm,
             cbuf, abuf, pop, confv, sem_c, sem_a):
    wid = lax.axis_index("s") * NC + lax.axis_index("c")
    r4 = wid * (ROWS_W // 128)
    cpa = pltpu.async_copy(
        arg_hbm.at[pl.ds(r4, ROWS_W // 128), :], abuf, sem_a)
    cpc = pltpu.async_copy(
        conf_hbm.at[pl.ds(r4, ROWS_W // 128), :], cbuf, sem_c)

    zeros16 = jnp.zeros((LANES,), jnp.float32)
    for k in range(N_COMP // LANES):
        pop[pl.ds(k * LANES, LANES)] = zeros16
    ones16 = jnp.ones((LANES,), jnp.float32)

    cpa.wait()
    for t in range(ROWS_W // LANES):
        idx = abuf[t // 8, pl.ds((t % 8) * LANES, LANES)]
        plsc.store_scatter(pop, [idx], ones16)
    cpc.wait()
    acc = zeros16
    for t in range(ROWS_W // LANES):
        acc = acc + cbuf[t // 8, pl.ds((t % 8) * LANES, LANES)]
    confv[...] = acc
    pltpu.sync_copy(pop, pop_hbm.at[wid])
    pltpu.sync_copy(confv, confp_hbm.at[wid])


def _merge_body(pop_ref, conf_ref, np_ref, cm_ref):
    present = jnp.max(pop_ref[...], axis=0, keepdims=True)
    num_pop = jnp.sum(jnp.where(present > 0.0, 1.0, 0.0))
    np_ref[...] = num_pop.reshape(1, 1)
    cm_ref[...] = (jnp.sum(conf_ref[...]) * (1.0 / ROWS)).reshape(1, 1)


_merge = pl.pallas_call(
    _merge_body,
    out_shape=(
        jax.ShapeDtypeStruct((1, 1), jnp.float32),
        jax.ShapeDtypeStruct((1, 1), jnp.float32),
    ),
)


def kernel(z, z_cat):
    conf2, arg2 = _prep(z_cat)
    pop_part, conf_part = _sc_hist(conf2, arg2)
    num_pop, conf_mean = _merge(pop_part, conf_part)
    return (z, num_pop[0, 0], conf_mean[0, 0])
